# Initial kernel scaffold; baseline (speedup 1.0000x reference)
import jax
import jax.numpy as jnp
from jax.experimental import pallas as pl


def kernel(x, W1, W2):
    raise NotImplementedError("write your pallas kernel here")


# speedup vs baseline: 1.0046x; 1.0046x over previous
import functools

import jax
import jax.numpy as jnp
from jax.experimental import pallas as pl

_NUM_EXPERTS = 16
_NUM_SELECTS = 2
_BLOCK = 512


def _gate_body(x_ref, w1t_ref, w2t_ref, r_ref, idx_ref, s_ref):
    xb = x_ref[...]
    h = jnp.tanh(
        jax.lax.dot_general(
            xb, w1t_ref[...], (((1,), (0,)), ((), ())),
            preferred_element_type=jnp.float32,
        )
    )
    logits = jax.lax.dot_general(
        h, w2t_ref[...], (((1,), (0,)), ((), ())),
        preferred_element_type=jnp.float32,
    )

    r = r_ref[...]
    iota = jax.lax.broadcasted_iota(jnp.int32, r.shape, 1)
    m0 = jnp.max(r, axis=1, keepdims=True)
    i0 = jnp.min(jnp.where(r == m0, iota, _NUM_EXPERTS), axis=1, keepdims=True)
    r2 = jnp.where(iota == i0, -1.0, r)
    m1 = jnp.max(r2, axis=1, keepdims=True)
    i1 = jnp.min(jnp.where(r2 == m1, iota, _NUM_EXPERTS), axis=1, keepdims=True)

    l0 = jnp.sum(jnp.where(iota == i0, logits, 0.0), axis=1, keepdims=True)
    l1 = jnp.sum(jnp.where(iota == i1, logits, 0.0), axis=1, keepdims=True)
    mx = jnp.maximum(l0, l1)
    e0 = jnp.exp(l0 - mx)
    e1 = jnp.exp(l1 - mx)
    denom = e0 + e1

    idx_ref[...] = jnp.concatenate([i0, i1], axis=1)
    s_ref[...] = jnp.concatenate([e0 / denom, e1 / denom], axis=1)


@functools.partial(jax.jit, static_argnames=())
def _gate(x, w1t, w2t, rand):
    n = x.shape[0]
    d = x.shape[1]
    grid = (n // _BLOCK,)
    idx, scores = pl.pallas_call(
        _gate_body,
        grid=grid,
        in_specs=[
            pl.BlockSpec((_BLOCK, d), lambda i: (i, 0)),
            pl.BlockSpec((d, _NUM_EXPERTS), lambda i: (0, 0)),
            pl.BlockSpec((_NUM_EXPERTS, _NUM_EXPERTS), lambda i: (0, 0)),
            pl.BlockSpec((_BLOCK, _NUM_EXPERTS), lambda i: (i, 0)),
        ],
        out_specs=[
            pl.BlockSpec((_BLOCK, _NUM_SELECTS), lambda i: (i, 0)),
            pl.BlockSpec((_BLOCK, _NUM_SELECTS), lambda i: (i, 0)),
        ],
        out_shape=[
            jax.ShapeDtypeStruct((n, _NUM_SELECTS), jnp.int32),
            jax.ShapeDtypeStruct((n, _NUM_SELECTS), jnp.float32),
        ],
    )(x, w1t, w2t, rand)
    return idx, scores


def kernel(x, W1, W2):
    n = x.shape[0]
    rand = jax.random.uniform(
        jax.random.key(42), (n, _NUM_EXPERTS), dtype=jnp.float32
    )
    idx, scores = _gate(x, W1.T, W2.T, rand)
    balance_loss = jnp.array(0, dtype=jnp.int32)
    load = jnp.array(-1, dtype=jnp.int32)
    importance = jnp.array(-1, dtype=jnp.int32)
    return idx, scores, balance_loss, load, importance


# speedup vs baseline: 1.0066x; 1.0020x over previous
import functools

import jax
import jax.numpy as jnp
from jax.experimental import pallas as pl

_NUM_EXPERTS = 16
_NUM_SELECTS = 2
_BLOCK = 512


def _gate_body(x_ref, w1t_ref, w2t_ref, r_ref, idx_ref, s_ref):
    xb = x_ref[...].astype(jnp.bfloat16)
    h = jnp.tanh(
        jax.lax.dot_general(
            xb, w1t_ref[...].astype(jnp.bfloat16), (((1,), (0,)), ((), ())),
            preferred_element_type=jnp.float32,
        )
    )
    logits = jax.lax.dot_general(
        h, w2t_ref[...], (((1,), (0,)), ((), ())),
        preferred_element_type=jnp.float32,
    )

    r = r_ref[...]
    iota = jax.lax.broadcasted_iota(jnp.int32, r.shape, 1)
    m0 = jnp.max(r, axis=1, keepdims=True)
    i0 = jnp.min(jnp.where(r == m0, iota, _NUM_EXPERTS), axis=1, keepdims=True)
    r2 = jnp.where(iota == i0, -1.0, r)
    m1 = jnp.max(r2, axis=1, keepdims=True)
    i1 = jnp.min(jnp.where(r2 == m1, iota, _NUM_EXPERTS), axis=1, keepdims=True)

    l0 = jnp.sum(jnp.where(iota == i0, logits, 0.0), axis=1, keepdims=True)
    l1 = jnp.sum(jnp.where(iota == i1, logits, 0.0), axis=1, keepdims=True)
    mx = jnp.maximum(l0, l1)
    e0 = jnp.exp(l0 - mx)
    e1 = jnp.exp(l1 - mx)
    denom = e0 + e1

    idx_ref[...] = jnp.concatenate([i0, i1], axis=1)
    s_ref[...] = jnp.concatenate([e0 / denom, e1 / denom], axis=1)


@functools.partial(jax.jit, static_argnames=())
def _gate(x, w1t, w2t, rand):
    n = x.shape[0]
    d = x.shape[1]
    grid = (n // _BLOCK,)
    idx, scores = pl.pallas_call(
        _gate_body,
        grid=grid,
        in_specs=[
            pl.BlockSpec((_BLOCK, d), lambda i: (i, 0)),
            pl.BlockSpec((d, _NUM_EXPERTS), lambda i: (0, 0)),
            pl.BlockSpec((_NUM_EXPERTS, _NUM_EXPERTS), lambda i: (0, 0)),
            pl.BlockSpec((_BLOCK, _NUM_EXPERTS), lambda i: (i, 0)),
        ],
        out_specs=[
            pl.BlockSpec((_BLOCK, _NUM_SELECTS), lambda i: (i, 0)),
            pl.BlockSpec((_BLOCK, _NUM_SELECTS), lambda i: (i, 0)),
        ],
        out_shape=[
            jax.ShapeDtypeStruct((n, _NUM_SELECTS), jnp.int32),
            jax.ShapeDtypeStruct((n, _NUM_SELECTS), jnp.float32),
        ],
    )(x, w1t, w2t, rand)
    return idx, scores


def kernel(x, W1, W2):
    n = x.shape[0]
    rand = jax.random.uniform(
        jax.random.key(42), (n, _NUM_EXPERTS), dtype=jnp.float32
    )
    idx, scores = _gate(x, W1.T, W2.T, rand)
    balance_loss = jnp.array(0, dtype=jnp.int32)
    load = jnp.array(-1, dtype=jnp.int32)
    importance = jnp.array(-1, dtype=jnp.int32)
    return idx, scores, balance_loss, load, importance
